# baseline (speedup 1.0000x reference)
import functools

import jax
import jax.numpy as jnp
from jax.experimental import pallas as pl
from jax.experimental.pallas import tpu as pltpu

_E = 16
_I = 16
_T = 10
_H = 64
_HL = 64
_EI = _E + _I
_TE = _T * _E
_O = 64
_T_FUTURE = 8
_G = 4
_GP = 4


def _mm(a, b):
    return jax.lax.dot_general(
        a.astype(jnp.bfloat16),
        b,
        (((1,), (0,)), ((), ())),
        preferred_element_type=jnp.float32,
    )


def _mm16(a, b):
    return _mm(a, b).astype(jnp.bfloat16)


def _lanecat(x):
    return jnp.concatenate([x[k * _O:(k + 1) * _O] for k in range(_G)], axis=1)


def _edges_agg(acs, w2, b2, w3, b3):
    n = _O
    zero = jnp.bfloat16(0)
    h1s = [
        jnp.maximum(
            a.astype(jnp.bfloat16)[:, None, :]
            + c.astype(jnp.bfloat16)[None, :, :],
            zero,
        ).reshape(n * n, -1)
        for a, c in acs
    ]
    m2s = [_mm(h1, w2) for h1 in h1s]
    h2s = [jnp.maximum(m2 + b2, 0.0).astype(jnp.bfloat16) for m2 in m2s]
    m3s = [_mm(h2, w3) for h2 in h2s]
    aggs = []
    for m3 in m3s:
        e3 = (m3 + b3).reshape(n, n, -1)
        ii = jax.lax.broadcasted_iota(jnp.int32, e3.shape, 0)
        jj = jax.lax.broadcasted_iota(jnp.int32, e3.shape, 1)
        me = jnp.where(jj > ii, e3, 0.0)
        aggs.append(me.sum(axis=1) - me.sum(axis=0))
    return aggs


def _body(
    src_ref, cur_ref,
    e_w1a, e_w1b, e_b1, e_w2, e_b2, e_w3, e_b3,
    n_w1a, n_w1b, n_b1, n_w2, n_b2, n_w3, n_b3, n_w4, n_b4,
    l_w1a, l_w1b, l_b1, l_w2, l_b2, l_w3, l_b3,
    t_w1a, t_w1b, t_b1, t_w2, t_b2, t_w3, t_b3, t_w4, t_b4, t_w5, t_b5,
    out_ref,
):
    srcs = [src_ref[gp * _G:(gp + 1) * _G].reshape(_G * _O, _TE)
            for gp in range(_GP)]

    acs = [(_lanecat(_mm(src, e_w1a[...])) + e_b1[...],
            _lanecat(_mm(src, e_w1b[...]))) for src in srcs]
    aggs = _edges_agg(acs, e_w2[...], e_b2[...], e_w3[...], e_b3[...])
    zs = []
    for gp in range(_GP):
        n_src = _lanecat(_mm(srcs[gp], n_w1a[...]))
        h = jnp.maximum(n_src + _mm(aggs[gp], n_w1b[...]) + n_b1[...], 0.0)
        h = jnp.maximum(_mm(h, n_w2[...]) + n_b2[...], 0.0)
        h = jnp.maximum(_mm(h, n_w3[...]) + n_b3[...], 0.0)
        z_impl = _mm(h, n_w4[...]) + n_b4[...]
        cur = _lanecat(cur_ref[gp * _G:(gp + 1) * _G].reshape(_G * _O, _E))
        zs.append(jnp.concatenate(
            sum(([cur[:, k * _E:(k + 1) * _E], z_impl[:, k * _I:(k + 1) * _I]]
                 for k in range(_G)), []),
            axis=1,
        ))

    for t in range(_T_FUTURE):
        acs = [(_mm(z, l_w1a[...]) + l_b1[...], _mm(z, l_w1b[...]))
               for z in zs]
        aggs = _edges_agg(acs, l_w2[...], l_b2[...], l_w3[...], l_b3[...])
        for gp in range(_GP):
            z = zs[gp]
            h = jnp.maximum(_mm(z, t_w1a[...]) + _mm(aggs[gp], t_w1b[...])
                            + t_b1[...], 0.0)
            h = jnp.maximum(_mm(h, t_w2[...]) + t_b2[...], 0.0)
            h = jnp.maximum(_mm(h, t_w3[...]) + t_b3[...], 0.0)
            h = jnp.maximum(_mm(h, t_w4[...]) + t_b4[...], 0.0)
            delta = _mm(h, t_w5[...]) + t_b5[...]
            z = z + delta
            zs[gp] = z
            for k in range(_G):
                out_ref[gp * _G + k, t] = z[:, k * _EI:k * _EI + _E]


def _run(source, current, weights, interpret=False):
    bb = source.shape[0]

    def full(w):
        return pl.BlockSpec(w.shape, lambda b: (0,) * w.ndim)

    gb = _G * _GP
    in_specs = [
        pl.BlockSpec((gb, _O, _TE), lambda b: (b, 0, 0)),
        pl.BlockSpec((gb, _O, _E), lambda b: (b, 0, 0)),
    ] + [full(w) for w in weights]

    return pl.pallas_call(
        _body,
        grid=(bb // gb,),
        in_specs=in_specs,
        out_specs=pl.BlockSpec((gb, _T_FUTURE, _O, _E), lambda b: (b, 0, 0, 0)),
        out_shape=jax.ShapeDtypeStruct((bb, _T_FUTURE, _O, _E), jnp.float32),
        compiler_params=pltpu.CompilerParams(
            dimension_semantics=("parallel",),
        ),
        interpret=interpret,
    )(source, current, *weights)


def _bd(w):
    k, n = w.shape
    out = jnp.zeros((_G * k, _G * n), w.dtype)
    for i in range(_G):
        out = out.at[i * k:(i + 1) * k, i * n:(i + 1) * n].set(w)
    return out.astype(jnp.bfloat16)


def _tile(b):
    return jnp.tile(b.reshape(1, -1), (1, _G)).astype(jnp.bfloat16)


def _pack(params, d0, bd_a, bd_b):
    (w1, b1), rest = params[0], params[1:]
    wa, wb = w1[:d0], w1[d0:]
    out = [_bd(wa) if bd_a else wa.astype(jnp.bfloat16),
           _bd(wb) if bd_b else wb.astype(jnp.bfloat16),
           _tile(b1)]
    for w, b in rest:
        out += [_bd(w), _tile(b)]
    return out


def kernel(z_explicit_seq, t_future, edge_params, node_params, ledge_params,
           trans_params, interpret=False):
    bb, tt, oo, ee = z_explicit_seq.shape
    diffs = z_explicit_seq[:, 1:] - z_explicit_seq[:, :-1]
    first = z_explicit_seq[:, 0:1]
    current = z_explicit_seq[:, -1]
    source = (
        jnp.concatenate([first, diffs], axis=1)
        .transpose(0, 2, 1, 3)
        .reshape(bb, oo, tt * ee)
    )
    weights = tuple(
        _pack(edge_params, _TE, False, False)
        + _pack(node_params, _TE, False, True)
        + _pack(ledge_params, _EI, True, True)
        + _pack(trans_params, _EI, True, True)
    )

    return _run(source, current, weights, interpret=interpret)


# speedup vs baseline: 1.2447x; 1.2447x over previous
import functools

import jax
import jax.numpy as jnp
from jax.experimental import pallas as pl
from jax.experimental.pallas import tpu as pltpu

_E = 16
_I = 16
_T = 10
_H = 64
_HL = 64
_EI = _E + _I
_TE = _T * _E
_O = 64
_T_FUTURE = 8
_G = 4
_GP = 2


def _mm(a, b):
    return jax.lax.dot_general(
        a.astype(jnp.bfloat16),
        b,
        (((1,), (0,)), ((), ())),
        preferred_element_type=jnp.float32,
    )


def _mm16(a, b):
    return _mm(a, b).astype(jnp.bfloat16)


def _lanecat(x):
    return jnp.concatenate([x[k * _O:(k + 1) * _O] for k in range(_G)], axis=1)


def _edges_agg(acs, w2, b2, w3, b3):
    n = _O
    zero = jnp.bfloat16(0)
    h1s = [
        jnp.maximum(
            a.astype(jnp.bfloat16)[:, None, :]
            + c.astype(jnp.bfloat16)[None, :, :],
            zero,
        ).reshape(n * n, -1)
        for a, c in acs
    ]
    m2s = [_mm(h1, w2) for h1 in h1s]
    gs = []
    for m2 in m2s:
        h2 = jnp.maximum(m2 + b2, 0.0).reshape(n, n, -1)
        ii = jax.lax.broadcasted_iota(jnp.int32, h2.shape, 0)
        jj = jax.lax.broadcasted_iota(jnp.int32, h2.shape, 1)
        me = jnp.where(jj > ii, h2, 0.0)
        gs.append(me.sum(axis=1) - me.sum(axis=0))
    io = jax.lax.broadcasted_iota(
        jnp.int32, (n, b3.shape[-1]), 0).astype(jnp.float32)
    cb = (float(n - 1) - 2.0 * io) * b3
    return [_mm(g, w3) + cb for g in gs]


def _body(
    src_ref, cur_ref,
    e_w1a, e_w1b, e_b1, e_w2, e_b2, e_w3, e_b3,
    n_w1a, n_w1b, n_b1, n_w2, n_b2, n_w3, n_b3, n_w4, n_b4,
    l_w1a, l_w1b, l_b1, l_w2, l_b2, l_w3, l_b3,
    t_w1a, t_w1b, t_b1, t_w2, t_b2, t_w3, t_b3, t_w4, t_b4, t_w5, t_b5,
    out_ref,
):
    srcs = [src_ref[gp * _G:(gp + 1) * _G].reshape(_G * _O, _TE)
            for gp in range(_GP)]

    acs = [(_lanecat(_mm(src, e_w1a[...])) + e_b1[...],
            _lanecat(_mm(src, e_w1b[...]))) for src in srcs]
    aggs = _edges_agg(acs, e_w2[...], e_b2[...], e_w3[...], e_b3[...])
    zs = []
    for gp in range(_GP):
        n_src = _lanecat(_mm(srcs[gp], n_w1a[...]))
        h = jnp.maximum(n_src + _mm(aggs[gp], n_w1b[...]) + n_b1[...], 0.0)
        h = jnp.maximum(_mm(h, n_w2[...]) + n_b2[...], 0.0)
        h = jnp.maximum(_mm(h, n_w3[...]) + n_b3[...], 0.0)
        z_impl = _mm(h, n_w4[...]) + n_b4[...]
        cur = _lanecat(cur_ref[gp * _G:(gp + 1) * _G].reshape(_G * _O, _E))
        zs.append(jnp.concatenate(
            sum(([cur[:, k * _E:(k + 1) * _E], z_impl[:, k * _I:(k + 1) * _I]]
                 for k in range(_G)), []),
            axis=1,
        ))

    for t in range(_T_FUTURE):
        acs = [(_mm(z, l_w1a[...]) + l_b1[...], _mm(z, l_w1b[...]))
               for z in zs]
        aggs = _edges_agg(acs, l_w2[...], l_b2[...], l_w3[...], l_b3[...])
        for gp in range(_GP):
            z = zs[gp]
            h = jnp.maximum(_mm(z, t_w1a[...]) + _mm(aggs[gp], t_w1b[...])
                            + t_b1[...], 0.0)
            h = jnp.maximum(_mm(h, t_w2[...]) + t_b2[...], 0.0)
            h = jnp.maximum(_mm(h, t_w3[...]) + t_b3[...], 0.0)
            h = jnp.maximum(_mm(h, t_w4[...]) + t_b4[...], 0.0)
            delta = _mm(h, t_w5[...]) + t_b5[...]
            z = z + delta
            zs[gp] = z
            for k in range(_G):
                out_ref[gp * _G + k, t] = z[:, k * _EI:k * _EI + _E]


def _run(source, current, weights, interpret=False):
    bb = source.shape[0]

    def full(w):
        return pl.BlockSpec(w.shape, lambda b: (0,) * w.ndim)

    gb = _G * _GP
    in_specs = [
        pl.BlockSpec((gb, _O, _TE), lambda b: (b, 0, 0)),
        pl.BlockSpec((gb, _O, _E), lambda b: (b, 0, 0)),
    ] + [full(w) for w in weights]

    return pl.pallas_call(
        _body,
        grid=(bb // gb,),
        in_specs=in_specs,
        out_specs=pl.BlockSpec((gb, _T_FUTURE, _O, _E), lambda b: (b, 0, 0, 0)),
        out_shape=jax.ShapeDtypeStruct((bb, _T_FUTURE, _O, _E), jnp.float32),
        compiler_params=pltpu.CompilerParams(
            dimension_semantics=("parallel",),
        ),
        interpret=interpret,
    )(source, current, *weights)


def _bd(w):
    k, n = w.shape
    out = jnp.zeros((_G * k, _G * n), w.dtype)
    for i in range(_G):
        out = out.at[i * k:(i + 1) * k, i * n:(i + 1) * n].set(w)
    return out.astype(jnp.bfloat16)


def _tile(b):
    return jnp.tile(b.reshape(1, -1), (1, _G)).astype(jnp.bfloat16)


def _pack(params, d0, bd_a, bd_b):
    (w1, b1), rest = params[0], params[1:]
    wa, wb = w1[:d0], w1[d0:]
    out = [_bd(wa) if bd_a else wa.astype(jnp.bfloat16),
           _bd(wb) if bd_b else wb.astype(jnp.bfloat16),
           _tile(b1)]
    for w, b in rest:
        out += [_bd(w), _tile(b)]
    return out


def kernel(z_explicit_seq, t_future, edge_params, node_params, ledge_params,
           trans_params, interpret=False):
    bb, tt, oo, ee = z_explicit_seq.shape
    diffs = z_explicit_seq[:, 1:] - z_explicit_seq[:, :-1]
    first = z_explicit_seq[:, 0:1]
    current = z_explicit_seq[:, -1]
    source = (
        jnp.concatenate([first, diffs], axis=1)
        .transpose(0, 2, 1, 3)
        .reshape(bb, oo, tt * ee)
    )
    weights = tuple(
        _pack(edge_params, _TE, False, False)
        + _pack(node_params, _TE, False, True)
        + _pack(ledge_params, _EI, True, True)
        + _pack(trans_params, _EI, True, True)
    )

    return _run(source, current, weights, interpret=interpret)
